# baseline (speedup 1.0000x reference)
import functools

import jax
import jax.numpy as jnp
from jax import lax
from jax.experimental import pallas as pl
from jax.experimental.pallas import tpu as pltpu
from jax.experimental.pallas import tpu_sc as plsc

_B, _S, _D, _V = 1024, 50, 512, 100000
_NC, _NS = 2, 16
_NW = _NC * _NS
_BPW = _B // _NW
_NCHUNK = _BPW
_CHUNK = _S
_LANES = 16
_H = 8
_T = _D // 128
_G = 7
_CR = _G * _T * _H
_IPW = _BPW * _S
_IDXPAD = 1664


def _pe_table():
    i = jnp.arange(_S, dtype=jnp.float32)[:, None]
    j = jnp.arange(_D // 2, dtype=jnp.float32)[None, :]
    ang = i / jnp.power(jnp.float32(10000.0), 2.0 * j / _D)
    pe = jnp.zeros((_S, _D), dtype=jnp.float32)
    pe = pe.at[:, 0::2].set(jnp.sin(ang))
    pe = pe.at[:, 1::2].set(jnp.cos(ang))
    pe_p = jnp.pad(pe, ((0, _G * _H - _S), (0, 0)))
    return (pe_p.reshape(_G, _H, _T, 128)
            .transpose(0, 2, 1, 3)
            .reshape(_CR, 128))


_mesh = plsc.VectorSubcoreMesh(core_axis_name="c", subcore_axis_name="s")


@functools.partial(
    pl.kernel,
    mesh=_mesh,
    out_type=jax.ShapeDtypeStruct((_B, _CR, 128), jnp.float32),
    scratch_types=[
        pltpu.VMEM((_IDXPAD,), jnp.int32),
        pltpu.VMEM((_NCHUNK, 2, 112), jnp.int32),
        pltpu.VMEM((_CR, 128), jnp.float32),
        pltpu.VMEM((2, _CR, 128), jnp.float32),
        pltpu.SemaphoreType.DMA,
        pltpu.SemaphoreType.DMA,
        pltpu.SemaphoreType.DMA,
        pltpu.SemaphoreType.DMA,
    ],
    compiler_params=pltpu.CompilerParams(use_tc_tiling_on_sc=False),
)
def _sc_lookup(x_hbm, pe_hbm, tbl_hbm, out_hbm, idx_v, seg_v, pe_v, rows_v,
               gsem0, gsem1, ssem0, ssem1):
    gsems = (gsem0, gsem1)
    ssems = (ssem0, ssem1)
    wid = lax.axis_index("s") * _NC + lax.axis_index("c")
    pltpu.sync_copy(x_hbm.at[wid], idx_v.at[pl.ds(0, _IPW)])
    pltpu.sync_copy(pe_hbm, pe_v)

    iota = lax.iota(jnp.int32, _LANES)
    lane_s = iota & 7

    def expand(j, _):
        for g in range(_G):
            row = (g * _H) + lane_s
            live = row < _CHUNK
            a = idx_v[pl.ds(j * _CHUNK + g * _H, _LANES)]
            gv = a.at[lane_s].get(mode="promise_in_bounds")
            base = ((gv >> 3) << 5) + (gv & 7)
            for k in range(2):
                t_lane = (iota >> 3) + 2 * k
                grp = 2 * g + k
                seg_v[j, grp // 7, pl.ds((grp % 7) * _LANES, _LANES)] = (
                    jnp.where(live, base + (t_lane << 3), 0))
        return _

    lax.fori_loop(0, _NCHUNK, expand, 0)

    def add_pe(i, buf):
        for r in range(2):
            row = 2 * i + r
            for q in range(8):
                sl = pl.ds(q * _LANES, _LANES)
                plsc.addupdate(rows_v.at[buf, row, sl], pe_v[row, sl])
        return buf

    def add_pe_tail(buf):
        for t in range(_T):
            for s in range(2):
                row = 6 * 32 + t * _H + s
                for q in range(8):
                    sl = pl.ds(q * _LANES, _LANES)
                    plsc.addupdate(rows_v.at[buf, row, sl], pe_v[row, sl])

    def issue_gathers(j, b):
        for h in range(2):
            pltpu.async_copy(
                tbl_hbm.at[seg_v.at[j, h]],
                rows_v.at[b, pl.ds(h * 112, 112)],
                gsems[b],
            )

    def drain_gathers(b):
        pltpu.make_async_copy(
            tbl_hbm.at[pl.ds(0, _CR)], rows_v.at[b], gsems[b]).wait()

    def drain_scatter(b):
        pltpu.make_async_copy(rows_v.at[b], out_hbm.at[0], ssems[b]).wait()

    issue_gathers(0, 0)

    def pair_body(j2, _):
        for sub in range(2):
            j = 2 * j2 + sub
            b = sub
            nb = 1 - sub
            drain_gathers(b)

            @pl.when(j >= 1)
            def _wait_prev_scatter():
                drain_scatter(nb)

            @pl.when(j + 1 < _NCHUNK)
            def _issue_next():
                issue_gathers(j + 1, nb)

            pltpu.async_copy(rows_v.at[b], out_hbm.at[wid * _NCHUNK + j],
                             ssems[b])
        return _

    lax.fori_loop(0, _NCHUNK // 2, pair_body, 0)
    drain_scatter(1)


def kernel(x, offsets, table):
    del offsets
    x3 = x.reshape(_NW, _IPW)
    t2 = (table.reshape(_V // _H, _H, _T, 128)
          .transpose(0, 2, 1, 3)
          .reshape(_V * _T, 128))
    out = _sc_lookup(x3, _pe_table(), t2)
    return (out.reshape(_B, _G, _T, _H, 128)
            .transpose(0, 1, 3, 2, 4)
            .reshape(_B, _G * _H, _D)[:, :_S, :])


# speedup vs baseline: 3.3792x; 3.3792x over previous
import functools

import jax
import jax.numpy as jnp
from jax import lax
from jax.experimental import pallas as pl
from jax.experimental.pallas import tpu as pltpu
from jax.experimental.pallas import tpu_sc as plsc

_B, _S, _D, _V = 1024, 50, 512, 100000
_NC, _NS = 2, 16
_NW = _NC * _NS
_BPW = _B // _NW
_NCHUNK = _BPW
_CHUNK = _S
_LANES = 16


def _pe_table():
    i = jnp.arange(_S, dtype=jnp.float32)[:, None]
    j = jnp.arange(_D // 2, dtype=jnp.float32)[None, :]
    ang = i / jnp.power(jnp.float32(10000.0), 2.0 * j / _D)
    pe = jnp.zeros((_S, _D), dtype=jnp.float32)
    pe = pe.at[:, 0::2].set(jnp.sin(ang))
    pe = pe.at[:, 1::2].set(jnp.cos(ang))
    return pe


_mesh = plsc.VectorSubcoreMesh(core_axis_name="c", subcore_axis_name="s")


@functools.partial(
    pl.kernel,
    mesh=_mesh,
    out_type=jax.ShapeDtypeStruct((_B, _S, _D), jnp.float32),
    scratch_types=[
        pltpu.VMEM((_NCHUNK, _CHUNK), jnp.int32),
        pltpu.VMEM((_S, _D), jnp.float32),
        pltpu.VMEM((2, _CHUNK, _D), jnp.float32),
        pltpu.SemaphoreType.DMA,
        pltpu.SemaphoreType.DMA,
        pltpu.SemaphoreType.DMA,
        pltpu.SemaphoreType.DMA,
    ],
    compiler_params=pltpu.CompilerParams(use_tc_tiling_on_sc=False),
)
def _sc_lookup(x_hbm, pe_hbm, tbl_hbm, out_hbm, idx_v, pe_v, rows_v,
               gsem0, gsem1, ssem0, ssem1):
    gsems = (gsem0, gsem1)
    ssems = (ssem0, ssem1)
    wid = lax.axis_index("s") * _NC + lax.axis_index("c")
    pltpu.sync_copy(x_hbm.at[wid], idx_v)
    pltpu.sync_copy(pe_hbm, pe_v)

    def add_pe(i, buf):
        vals = []
        for c in range(_D // _LANES):
            sl = pl.ds(c * _LANES, _LANES)
            vals.append((sl, pe_v[i, sl]))
        for sl, v in vals:
            plsc.addupdate(rows_v.at[buf, i, sl], v)
        return buf

    def issue_gather(j, b):
        pltpu.async_copy(tbl_hbm.at[idx_v.at[j]], rows_v.at[b], gsems[b])

    def drain_gather(b):
        pltpu.make_async_copy(
            tbl_hbm.at[pl.ds(0, _CHUNK)], rows_v.at[b], gsems[b]).wait()

    def drain_scatter(b):
        pltpu.make_async_copy(rows_v.at[b], out_hbm.at[0], ssems[b]).wait()

    issue_gather(0, 0)

    def pair_body(j2, _):
        for sub in range(2):
            j = 2 * j2 + sub
            b = sub
            nb = 1 - sub
            drain_gather(b)

            @pl.when(j >= 1)
            def _wait_prev_scatter():
                drain_scatter(nb)

            @pl.when(j + 1 < _NCHUNK)
            def _issue_next():
                issue_gather(j + 1, nb)

            lax.fori_loop(0, _CHUNK, add_pe, b)
            pltpu.async_copy(rows_v.at[b], out_hbm.at[wid * _NCHUNK + j],
                             ssems[b])
        return _

    lax.fori_loop(0, _NCHUNK // 2, pair_body, 0)
    drain_scatter(1)


def kernel(x, offsets, table):
    del offsets
    x3 = x.reshape(_NW, _NCHUNK, _CHUNK)
    return _sc_lookup(x3, _pe_table(), table)
